# baseline (speedup 1.0000x reference)
import functools

import jax
import jax.numpy as jnp
import numpy as np
from jax import lax
from jax.experimental import pallas as pl
from jax.experimental.pallas import tpu as pltpu
from jax.experimental.pallas import tpu_sc as plsc

T = 1024
N = 64
NMASK = 768
L = 16
NV = T // L
NWORKERS = 32

def _threefry2x32_np(k1, k2, x0, x1):
  rot = [(13, 15, 26, 6), (17, 29, 16, 24)]
  ks = [np.uint32(k1), np.uint32(k2),
        np.uint32(k1) ^ np.uint32(k2) ^ np.uint32(0x1BD11BDA)]
  x = [x0.astype(np.uint32), x1.astype(np.uint32)]

  def rnd(x, r):
    a = (x[0] + x[1]).astype(np.uint32)
    b = ((x[1] << np.uint32(r)) | (x[1] >> np.uint32(32 - r))).astype(np.uint32)
    return [a, a ^ b]

  x[0] = (x[0] + ks[0]).astype(np.uint32)
  x[1] = (x[1] + ks[1]).astype(np.uint32)
  for rs, a, b, i in [(rot[0], ks[1], ks[2], 1), (rot[1], ks[2], ks[0], 2),
                      (rot[0], ks[0], ks[1], 3), (rot[1], ks[1], ks[2], 4),
                      (rot[0], ks[2], ks[0], 5)]:
    for r in rs:
      x = rnd(x, r)
    x[0] = (x[0] + a).astype(np.uint32)
    x[1] = (x[1] + b + np.uint32(i)).astype(np.uint32)
  return x


def _build_ukeys_t_flat():
  idx64 = np.arange(T * N, dtype=np.uint64)
  b1, b2 = _threefry2x32_np(0, 42, (idx64 >> np.uint64(32)).astype(np.uint32),
                            (idx64 & np.uint64(0xFFFFFFFF)).astype(np.uint32))
  m = ((b1 ^ b2) >> np.uint32(9)).reshape(T, N)
  ukeys = (m << 9) | (np.arange(T, dtype=np.uint32)[:, None] >> 1)
  return np.ascontiguousarray(ukeys.T).reshape(-1)


_UKEYS_T_FLAT = _build_ukeys_t_flat()


def _sc_argsort_cols(ukeys_hbm, masked_hbm, unmasked_hbm, keys_v, vals_v):
  wid = lax.axis_index("s") * 2 + lax.axis_index("c")
  col_a = wid
  col_b = wid + NWORKERS

  def rev(v16):
    return lax.rev(v16, dimensions=(0,))

  def intra_first():
    def body(p, carry):
      for v in (2 * p, 2 * p + 1):
        base = v * L
        iv = jnp.arange(L, dtype=jnp.int32) + base
        for half in (0, T):
          sl = pl.ds(half + base, L)
          sk, sv = plsc.sort_key_val(keys_v[sl], iv)
          keys_v[sl] = sk
          vals_v[sl] = sv
      return carry
    lax.fori_loop(0, NV // 2, body, 0)

  def intra():
    def body(p, carry):
      for v in (2 * p, 2 * p + 1):
        base = v * L
        for half in (0, T):
          sl = pl.ds(half + base, L)
          sk, sv = plsc.sort_key_val(keys_v[sl], vals_v[sl])
          keys_v[sl] = sk
          vals_v[sl] = sv
      return carry
    lax.fori_loop(0, NV // 2, body, 0)

  def mirror(kv):
    half = kv // 2

    def body(p, carry):
      low = p & (half - 1)
      blk = (p - low) << 1
      v = blk + low
      w = blk + kv - 1 - low
      for hoff in (0, T):
        sa = pl.ds(hoff + v * L, L)
        sb = pl.ds(hoff + w * L, L)
        ka, kb = keys_v[sa], rev(keys_v[sb])
        va, vb = vals_v[sa], rev(vals_v[sb])
        sel = ka <= kb
        keys_v[sa] = jnp.where(sel, ka, kb)
        keys_v[sb] = rev(jnp.where(sel, kb, ka))
        vals_v[sa] = jnp.where(sel, va, vb)
        vals_v[sb] = rev(jnp.where(sel, vb, va))
      return carry
    lax.fori_loop(0, NV // 2, body, 0)

  def cross(jv):
    def body(p, carry):
      low = p & (jv - 1)
      v = ((p - low) << 1) | low
      w = v + jv
      for half in (0, T):
        sa = pl.ds(half + v * L, L)
        sb = pl.ds(half + w * L, L)
        ka, kb = keys_v[sa], keys_v[sb]
        va, vb = vals_v[sa], vals_v[sb]
        sel = ka <= kb
        keys_v[sa] = jnp.where(sel, ka, kb)
        keys_v[sb] = jnp.where(sel, kb, ka)
        vals_v[sa] = jnp.where(sel, va, vb)
        vals_v[sb] = jnp.where(sel, vb, va)
      return carry
    lax.fori_loop(0, NV // 2, body, 0)

  def cross1_intra():
    def body(p, carry):
      v = 2 * p
      for half in (0, T):
        sa = pl.ds(half + v * L, L)
        sb = pl.ds(half + (v + 1) * L, L)
        ka, kb = keys_v[sa], keys_v[sb]
        va, vb = vals_v[sa], vals_v[sb]
        sel = ka <= kb
        sk1, sv1 = plsc.sort_key_val(jnp.where(sel, ka, kb),
                                     jnp.where(sel, va, vb))
        sk2, sv2 = plsc.sort_key_val(jnp.where(sel, kb, ka),
                                     jnp.where(sel, vb, va))
        keys_v[sa] = sk1
        vals_v[sa] = sv1
        keys_v[sb] = sk2
        vals_v[sb] = sv2
      return carry
    lax.fori_loop(0, NV // 2, body, 0)

  def mirror2_intra():
    def body(p, carry):
      v = 2 * p
      for half in (0, T):
        sa = pl.ds(half + v * L, L)
        sb = pl.ds(half + (v + 1) * L, L)
        ka, kb = keys_v[sa], rev(keys_v[sb])
        va, vb = vals_v[sa], rev(vals_v[sb])
        sel = ka <= kb
        sk1, sv1 = plsc.sort_key_val(jnp.where(sel, ka, kb),
                                     jnp.where(sel, va, vb))
        sk2, sv2 = plsc.sort_key_val(jnp.where(sel, kb, ka),
                                     jnp.where(sel, vb, va))
        keys_v[sa] = sk1
        vals_v[sa] = sv1
        keys_v[sb] = sk2
        vals_v[sb] = sv2
      return carry
    lax.fori_loop(0, NV // 2, body, 0)

  def tail4():
    def cmpex(ka, kb, va, vb):
      sel = ka <= kb
      return (jnp.where(sel, ka, kb), jnp.where(sel, kb, ka),
              jnp.where(sel, va, vb), jnp.where(sel, vb, va))

    def body(p, carry):
      v = 4 * p
      for half in (0, T):
        sl = [pl.ds(half + (v + i) * L, L) for i in range(4)]
        k = [keys_v[s] for s in sl]
        w = [vals_v[s] for s in sl]
        k[0], k[2], w[0], w[2] = cmpex(k[0], k[2], w[0], w[2])
        k[1], k[3], w[1], w[3] = cmpex(k[1], k[3], w[1], w[3])
        k[0], k[1], w[0], w[1] = cmpex(k[0], k[1], w[0], w[1])
        k[2], k[3], w[2], w[3] = cmpex(k[2], k[3], w[2], w[3])
        for i in range(4):
          sk, sv = plsc.sort_key_val(k[i], w[i])
          keys_v[sl[i]] = sk
          vals_v[sl[i]] = sv
      return carry
    lax.fori_loop(0, NV // 4, body, 0)

  def tail8(hoff):
    def cmpex(ka, kb, va, vb):
      sel = ka <= kb
      return (jnp.where(sel, ka, kb), jnp.where(sel, kb, ka),
              jnp.where(sel, va, vb), jnp.where(sel, vb, va))

    def body(p, carry):
      v = 8 * p
      sl = [pl.ds(hoff + (v + i) * L, L) for i in range(8)]
      k = [keys_v[s] for s in sl]
      w = [vals_v[s] for s in sl]
      for a, b in ((0, 4), (1, 5), (2, 6), (3, 7),
                   (0, 2), (1, 3), (4, 6), (5, 7),
                   (0, 1), (2, 3), (4, 5), (6, 7)):
        k[a], k[b], w[a], w[b] = cmpex(k[a], k[b], w[a], w[b])
      for i in range(8):
        sk, sv = plsc.sort_key_val(k[i], w[i])
        keys_v[sl[i]] = sk
        vals_v[sl[i]] = sv
      return carry
    lax.fori_loop(0, NV // 8, body, 0)

  pltpu.sync_copy(ukeys_hbm.at[pl.ds(col_a * T, T)], keys_v.at[pl.ds(0, T)])
  pltpu.sync_copy(ukeys_hbm.at[pl.ds(col_b * T, T)], keys_v.at[pl.ds(T, T)])
  intra_first()
  mirror2_intra()
  mirror(4)
  cross1_intra()
  mirror(8)
  tail4()
  for kv in (16, 32, 64):
    mirror(kv)
    jv = kv // 4
    while jv >= 8:
      cross(jv)
      jv //= 2
    tail8(0)
    tail8(T)
  pltpu.sync_copy(vals_v.at[pl.ds(0, NMASK)], masked_hbm.at[col_a])
  pltpu.sync_copy(vals_v.at[pl.ds(NMASK, T - NMASK)], unmasked_hbm.at[col_a])
  pltpu.sync_copy(vals_v.at[pl.ds(T, NMASK)], masked_hbm.at[col_b])
  pltpu.sync_copy(vals_v.at[pl.ds(T + NMASK, T - NMASK)],
                  unmasked_hbm.at[col_b])


_sc_argsort = functools.partial(
    pl.kernel,
    out_type=(jax.ShapeDtypeStruct((N, NMASK), jnp.int32),
              jax.ShapeDtypeStruct((N, T - NMASK), jnp.int32)),
    mesh=plsc.VectorSubcoreMesh(core_axis_name="c", subcore_axis_name="s"),
    scratch_types=[pltpu.VMEM((2 * T,), jnp.uint32),
                   pltpu.VMEM((2 * T,), jnp.int32)],
    compiler_params=pltpu.CompilerParams(needs_layout_passes=False),
)(_sc_argsort_cols)


def kernel(x):
  del x
  masked_t, unmasked_t = _sc_argsort(jnp.asarray(_UKEYS_T_FLAT))
  return masked_t.T, unmasked_t.T


# speedup vs baseline: 1.0049x; 1.0049x over previous
import functools

import jax
import jax.numpy as jnp
import numpy as np
from jax import lax
from jax.experimental import pallas as pl
from jax.experimental.pallas import tpu as pltpu
from jax.experimental.pallas import tpu_sc as plsc

T = 1024
N = 64
NMASK = 768
L = 16
NV = T // L
NWORKERS = 32

def _threefry2x32_np(k1, k2, x0, x1):
  rot = [(13, 15, 26, 6), (17, 29, 16, 24)]
  ks = [np.uint32(k1), np.uint32(k2),
        np.uint32(k1) ^ np.uint32(k2) ^ np.uint32(0x1BD11BDA)]
  x = [x0.astype(np.uint32), x1.astype(np.uint32)]

  def rnd(x, r):
    a = (x[0] + x[1]).astype(np.uint32)
    b = ((x[1] << np.uint32(r)) | (x[1] >> np.uint32(32 - r))).astype(np.uint32)
    return [a, a ^ b]

  x[0] = (x[0] + ks[0]).astype(np.uint32)
  x[1] = (x[1] + ks[1]).astype(np.uint32)
  for rs, a, b, i in [(rot[0], ks[1], ks[2], 1), (rot[1], ks[2], ks[0], 2),
                      (rot[0], ks[0], ks[1], 3), (rot[1], ks[1], ks[2], 4),
                      (rot[0], ks[2], ks[0], 5)]:
    for r in rs:
      x = rnd(x, r)
    x[0] = (x[0] + a).astype(np.uint32)
    x[1] = (x[1] + b + np.uint32(i)).astype(np.uint32)
  return x


def _build_ukeys_t_flat():
  idx64 = np.arange(T * N, dtype=np.uint64)
  b1, b2 = _threefry2x32_np(0, 42, (idx64 >> np.uint64(32)).astype(np.uint32),
                            (idx64 & np.uint64(0xFFFFFFFF)).astype(np.uint32))
  m = ((b1 ^ b2) >> np.uint32(9)).reshape(T, N)
  ukeys = (m << 9) | (np.arange(T, dtype=np.uint32)[:, None] >> 1)
  return np.ascontiguousarray(ukeys.T).reshape(-1)


_UKEYS_T_FLAT = _build_ukeys_t_flat()


def _sc_argsort_cols(ukeys_hbm, masked_hbm, unmasked_hbm, keys_v, vals_v):
  wid = lax.axis_index("s") * 2 + lax.axis_index("c")
  col_a = wid
  col_b = wid + NWORKERS

  def rev(v16):
    return lax.rev(v16, dimensions=(0,))

  def intra_first():
    def body(p, carry):
      for v in (2 * p, 2 * p + 1):
        base = v * L
        iv = jnp.arange(L, dtype=jnp.int32) + base
        for half in (0, T):
          sl = pl.ds(half + base, L)
          sk, sv = plsc.sort_key_val(keys_v[sl], iv)
          keys_v[sl] = sk
          vals_v[sl] = sv
      return carry
    lax.fori_loop(0, NV // 2, body, 0)

  def intra():
    def body(p, carry):
      for v in (2 * p, 2 * p + 1):
        base = v * L
        for half in (0, T):
          sl = pl.ds(half + base, L)
          sk, sv = plsc.sort_key_val(keys_v[sl], vals_v[sl])
          keys_v[sl] = sk
          vals_v[sl] = sv
      return carry
    lax.fori_loop(0, NV // 2, body, 0)

  def mirror(kv):
    half = kv // 2

    def body(p, carry):
      low = p & (half - 1)
      blk = (p - low) << 1
      v = blk + low
      w = blk + kv - 1 - low
      for hoff in (0, T):
        sa = pl.ds(hoff + v * L, L)
        sb = pl.ds(hoff + w * L, L)
        ka, kb = keys_v[sa], rev(keys_v[sb])
        va, vb = vals_v[sa], rev(vals_v[sb])
        sel = ka <= kb
        keys_v[sa] = jnp.where(sel, ka, kb)
        keys_v[sb] = rev(jnp.where(sel, kb, ka))
        vals_v[sa] = jnp.where(sel, va, vb)
        vals_v[sb] = rev(jnp.where(sel, vb, va))
      return carry
    lax.fori_loop(0, NV // 2, body, 0)

  def cross(jv):
    def body(p, carry):
      low = p & (jv - 1)
      v = ((p - low) << 1) | low
      w = v + jv
      for half in (0, T):
        sa = pl.ds(half + v * L, L)
        sb = pl.ds(half + w * L, L)
        ka, kb = keys_v[sa], keys_v[sb]
        va, vb = vals_v[sa], vals_v[sb]
        sel = ka <= kb
        keys_v[sa] = jnp.where(sel, ka, kb)
        keys_v[sb] = jnp.where(sel, kb, ka)
        vals_v[sa] = jnp.where(sel, va, vb)
        vals_v[sb] = jnp.where(sel, vb, va)
      return carry
    lax.fori_loop(0, NV // 2, body, 0)

  def cross1_intra():
    def body(p, carry):
      v = 2 * p
      for half in (0, T):
        sa = pl.ds(half + v * L, L)
        sb = pl.ds(half + (v + 1) * L, L)
        ka, kb = keys_v[sa], keys_v[sb]
        va, vb = vals_v[sa], vals_v[sb]
        sel = ka <= kb
        sk1, sv1 = plsc.sort_key_val(jnp.where(sel, ka, kb),
                                     jnp.where(sel, va, vb))
        sk2, sv2 = plsc.sort_key_val(jnp.where(sel, kb, ka),
                                     jnp.where(sel, vb, va))
        keys_v[sa] = sk1
        vals_v[sa] = sv1
        keys_v[sb] = sk2
        vals_v[sb] = sv2
      return carry
    lax.fori_loop(0, NV // 2, body, 0)

  def mirror2_intra():
    def body(p, carry):
      v = 2 * p
      for half in (0, T):
        sa = pl.ds(half + v * L, L)
        sb = pl.ds(half + (v + 1) * L, L)
        ka, kb = keys_v[sa], rev(keys_v[sb])
        va, vb = vals_v[sa], rev(vals_v[sb])
        sel = ka <= kb
        sk1, sv1 = plsc.sort_key_val(jnp.where(sel, ka, kb),
                                     jnp.where(sel, va, vb))
        sk2, sv2 = plsc.sort_key_val(jnp.where(sel, kb, ka),
                                     jnp.where(sel, vb, va))
        keys_v[sa] = sk1
        vals_v[sa] = sv1
        keys_v[sb] = sk2
        vals_v[sb] = sv2
      return carry
    lax.fori_loop(0, NV // 2, body, 0)

  def tail4():
    def cmpex(ka, kb, va, vb):
      sel = ka <= kb
      return (jnp.where(sel, ka, kb), jnp.where(sel, kb, ka),
              jnp.where(sel, va, vb), jnp.where(sel, vb, va))

    def body(p, carry):
      v = 4 * p
      for half in (0, T):
        sl = [pl.ds(half + (v + i) * L, L) for i in range(4)]
        k = [keys_v[s] for s in sl]
        w = [vals_v[s] for s in sl]
        k[0], k[2], w[0], w[2] = cmpex(k[0], k[2], w[0], w[2])
        k[1], k[3], w[1], w[3] = cmpex(k[1], k[3], w[1], w[3])
        k[0], k[1], w[0], w[1] = cmpex(k[0], k[1], w[0], w[1])
        k[2], k[3], w[2], w[3] = cmpex(k[2], k[3], w[2], w[3])
        for i in range(4):
          sk, sv = plsc.sort_key_val(k[i], w[i])
          keys_v[sl[i]] = sk
          vals_v[sl[i]] = sv
      return carry
    lax.fori_loop(0, NV // 4, body, 0)

  pltpu.sync_copy(ukeys_hbm.at[pl.ds(col_a * T, T)], keys_v.at[pl.ds(0, T)])
  pltpu.sync_copy(ukeys_hbm.at[pl.ds(col_b * T, T)], keys_v.at[pl.ds(T, T)])
  intra_first()
  mirror2_intra()
  mirror(4)
  cross1_intra()
  for kv in (8, 16, 32, 64):
    mirror(kv)
    jv = kv // 4
    while jv >= 4:
      cross(jv)
      jv //= 2
    tail4()
  pltpu.sync_copy(vals_v.at[pl.ds(0, NMASK)], masked_hbm.at[col_a])
  pltpu.sync_copy(vals_v.at[pl.ds(NMASK, T - NMASK)], unmasked_hbm.at[col_a])
  pltpu.sync_copy(vals_v.at[pl.ds(T, NMASK)], masked_hbm.at[col_b])
  pltpu.sync_copy(vals_v.at[pl.ds(T + NMASK, T - NMASK)],
                  unmasked_hbm.at[col_b])


_sc_argsort = functools.partial(
    pl.kernel,
    out_type=(jax.ShapeDtypeStruct((N, NMASK), jnp.int32),
              jax.ShapeDtypeStruct((N, T - NMASK), jnp.int32)),
    mesh=plsc.VectorSubcoreMesh(core_axis_name="c", subcore_axis_name="s"),
    scratch_types=[pltpu.VMEM((2 * T,), jnp.uint32),
                   pltpu.VMEM((2 * T,), jnp.int32)],
    compiler_params=pltpu.CompilerParams(needs_layout_passes=False),
)(_sc_argsort_cols)


def kernel(x):
  del x
  masked_t, unmasked_t = _sc_argsort(jnp.asarray(_UKEYS_T_FLAT))
  return masked_t.T, unmasked_t.T
